# Initial kernel scaffold; baseline (speedup 1.0000x reference)
import jax
import jax.numpy as jnp
from jax.experimental import pallas as pl


def kernel(clips, pred_boxes, roi_features, pred_scores, pred_boxes_mask, w_bb, b_bb, wc, wr, wg, b1, wcls, bcls):
    raise NotImplementedError("write your pallas kernel here")


# speedup vs baseline: 1.7768x; 1.7768x over previous
import jax
import jax.numpy as jnp
from jax.experimental import pallas as pl
from jax.experimental.pallas import tpu as pltpu

_VMEM_LIMIT = 40 * 1024 * 1024
_NUM_CLASSES = 80


def _backbone_ctx(clips, w_bb, b_bb, hw_tile=1024):
    B, C, T, Hs, Ws = clips.shape
    HW = Hs * Ws
    Cf = w_bb.shape[0]
    n_tok = T * HW
    x3 = clips.reshape(B, C * T, HW)

    hw_tile = min(hw_tile, HW)
    assert HW % hw_tile == 0, (HW, hw_tile)
    n_hw = HW // hw_tile
    inv_n_tok = 1.0 / float(n_tok)

    eye_t = jnp.eye(T, dtype=w_bb.dtype)
    w_big = jnp.einsum("fc,ts->tfcs", w_bb, eye_t).reshape(T * Cf, C * T)
    nb_big = -jnp.tile(b_bb, (T, 1))

    def body(w_ref, nb_ref, b_ref, x_ref, out_ref, acc_ref):
        j = pl.program_id(1)
        feat = jnp.dot(w_ref[...], x_ref[0],
                       preferred_element_type=jnp.float32)
        feat = jnp.maximum(feat, nb_ref[...])
        folded = feat.reshape(T, Cf, hw_tile).sum(axis=0)

        @pl.when(j == 0)
        def _init():
            acc_ref[...] = jnp.zeros_like(acc_ref)

        acc_ref[...] += folded

        @pl.when(j == n_hw - 1)
        def _finalize():
            out_ref[0] = (jnp.sum(acc_ref[...], axis=1, keepdims=True)
                          * inv_n_tok + b_ref[...])

    out = pl.pallas_call(
        body,
        out_shape=jax.ShapeDtypeStruct((B, Cf, 1), jnp.float32),
        grid=(B, n_hw),
        in_specs=[
            pl.BlockSpec((T * Cf, C * T), lambda b, j: (0, 0)),
            pl.BlockSpec((T * Cf, 1), lambda b, j: (0, 0)),
            pl.BlockSpec((Cf, 1), lambda b, j: (0, 0)),
            pl.BlockSpec((1, C * T, hw_tile), lambda b, j: (b, 0, j)),
        ],
        out_specs=pl.BlockSpec((1, Cf, 1), lambda b, j: (b, 0, 0)),
        scratch_shapes=[pltpu.VMEM((Cf, hw_tile), jnp.float32)],
        compiler_params=pltpu.CompilerParams(
            dimension_semantics=("parallel", "arbitrary"),
            vmem_limit_bytes=_VMEM_LIMIT,
        ),
    )(w_big, nb_big, b_bb, x3)
    return out[:, :, 0]


def _head(ctx, roi2, pb8, mask_col, wr, wg8, wc, b1, wcls, bcls, nbox):
    M, Dr = roi2.shape
    B, Cf = ctx.shape
    Hh = wr.shape[1]
    ncls_pad = wcls.shape[1]
    n_split = 2
    half = M // n_split
    b_half = B // n_split

    def body(ctx_ref, roi_ref, pb_ref, m_ref, wr_ref, wg_ref, wc_ref,
             b1_ref, wcls_ref, bcls_ref, out_ref):
        i = pl.program_id(0)
        ctxh = jnp.dot(ctx_ref[...], wc_ref[...],
                       preferred_element_type=jnp.float32) + b1_ref[...]
        row_b = jax.lax.broadcasted_iota(jnp.int32, (half, B), 0) // nbox
        col_b = jax.lax.broadcasted_iota(jnp.int32, (half, B), 1)
        sel = (row_b + i * b_half == col_b).astype(jnp.float32)
        h = jnp.dot(roi_ref[...], wr_ref[...],
                    preferred_element_type=jnp.float32)
        h = h + jnp.dot(pb_ref[...], wg_ref[...],
                        preferred_element_type=jnp.float32)
        h = h + jnp.dot(sel, ctxh, preferred_element_type=jnp.float32)
        h = jnp.maximum(h, 0.0)
        logits = jnp.dot(h, wcls_ref[...],
                         preferred_element_type=jnp.float32) + bcls_ref[...]
        out_ref[...] = logits * m_ref[...]

    return pl.pallas_call(
        body,
        out_shape=jax.ShapeDtypeStruct((M, ncls_pad), jnp.float32),
        grid=(n_split,),
        in_specs=[
            pl.BlockSpec((B, Cf), lambda i: (0, 0)),
            pl.BlockSpec((half, Dr), lambda i: (i, 0)),
            pl.BlockSpec((half, 8), lambda i: (i, 0)),
            pl.BlockSpec((half, 1), lambda i: (i, 0)),
            pl.BlockSpec((Dr, Hh), lambda i: (0, 0)),
            pl.BlockSpec((8, Hh), lambda i: (0, 0)),
            pl.BlockSpec((Cf, Hh), lambda i: (0, 0)),
            pl.BlockSpec((1, Hh), lambda i: (0, 0)),
            pl.BlockSpec((Hh, ncls_pad), lambda i: (0, 0)),
            pl.BlockSpec((1, ncls_pad), lambda i: (0, 0)),
        ],
        out_specs=pl.BlockSpec((half, ncls_pad), lambda i: (i, 0)),
        compiler_params=pltpu.CompilerParams(
            dimension_semantics=("parallel",),
            vmem_limit_bytes=_VMEM_LIMIT,
        ),
    )(ctx, roi2, pb8, mask_col, wr, wg8, wc, b1, wcls, bcls)


def kernel(clips, pred_boxes, roi_features, pred_scores, pred_boxes_mask,
           w_bb, b_bb, wc, wr, wg, b1, wcls, bcls):
    B, nbox, dr = roi_features.shape
    ncls_pad = wcls.shape[1]
    M = B * nbox

    ctx = _backbone_ctx(clips, w_bb, b_bb)

    gmap = jnp.array(
        [[1, 0, 0, 0, 0, -1],
         [0, 1, 0, 0, -1, 0],
         [0, 0, 1, 0, 0, 1],
         [0, 0, 0, 1, 1, 0]], dtype=wg.dtype)
    wg_eff = gmap @ wg
    wg8 = jnp.concatenate(
        [wg_eff, jnp.zeros((4, wg.shape[1]), wg.dtype)], axis=0)
    pb8 = jnp.concatenate(
        [pred_boxes.reshape(M, 4),
         jnp.zeros((M, 4), pred_boxes.dtype)], axis=1)
    roi2 = roi_features.reshape(M, dr)
    mask_col = pred_boxes_mask.astype(jnp.float32).reshape(M, 1)

    logits = _head(ctx, roi2, pb8, mask_col, wr, wg8, wc, b1, wcls, bcls,
                   nbox)
    pred_logits = logits.reshape(B, nbox, ncls_pad)[:, :, :_NUM_CLASSES]

    outputs = {
        "pred_logits": pred_logits,
        "pred_scores": pred_scores,
        "pred_boxes_mask": pred_boxes_mask,
        "pred_boxes": pred_boxes,
    }
    return {"outputs": outputs}


# speedup vs baseline: 2.8019x; 1.5770x over previous
import jax
import jax.numpy as jnp
from jax.experimental import pallas as pl
from jax.experimental.pallas import tpu as pltpu

_VMEM_LIMIT = 40 * 1024 * 1024
_NUM_CLASSES = 80


def _backbone_ctx(clips, w_bb, b_bb, hw_tile=4096):
    B, C, T, Hs, Ws = clips.shape
    HW = Hs * Ws
    Cf = w_bb.shape[0]
    n_tok = T * HW
    x3 = clips.reshape(B, C * T, HW)

    hw_tile = min(hw_tile, HW)
    assert HW % hw_tile == 0, (HW, hw_tile)
    n_hw = HW // hw_tile
    inv_n_tok = 1.0 / float(n_tok)

    eye_t = jnp.eye(T, dtype=w_bb.dtype)
    w_big = jnp.einsum("fc,ts->tfcs", w_bb, eye_t).reshape(T * Cf, C * T)
    nb_big = -jnp.tile(b_bb, (T, 1))

    f8 = jnp.float8_e4m3fn
    w8 = w_big.astype(f8)
    w_res = (w_big - w8.astype(jnp.float32)).astype(f8)
    wcat = jnp.concatenate([w8, w_res, w8, w_res], axis=1)

    def _feat_folded(w_ref, nb_ref, x_ref):
        xf = x_ref[0]
        x8 = xf.astype(f8)
        x_res = (xf - x8.astype(jnp.float32)).astype(f8)
        xcat = jnp.concatenate([x8, x8, x_res, x_res], axis=0)
        feat = jnp.dot(w_ref[...], xcat,
                       preferred_element_type=jnp.float32)
        feat = jnp.maximum(feat, nb_ref[...])
        return feat.reshape(T, Cf, hw_tile).sum(axis=0)

    def body_single(w_ref, nb_ref, b_ref, x_ref, out_ref):
        folded = _feat_folded(w_ref, nb_ref, x_ref)
        out_ref[0] = (jnp.sum(folded, axis=1, keepdims=True)
                      * inv_n_tok + b_ref[...])

    def body_multi(w_ref, nb_ref, b_ref, x_ref, out_ref, acc_ref):
        j = pl.program_id(1)
        folded = _feat_folded(w_ref, nb_ref, x_ref)

        @pl.when(j == 0)
        def _init():
            acc_ref[...] = jnp.zeros_like(acc_ref)

        acc_ref[...] += folded

        @pl.when(j == n_hw - 1)
        def _finalize():
            out_ref[0] = (jnp.sum(acc_ref[...], axis=1, keepdims=True)
                          * inv_n_tok + b_ref[...])

    out = pl.pallas_call(
        body_single if n_hw == 1 else body_multi,
        out_shape=jax.ShapeDtypeStruct((B, Cf, 1), jnp.float32),
        grid=(B, n_hw),
        in_specs=[
            pl.BlockSpec((T * Cf, 4 * C * T), lambda b, j: (0, 0)),
            pl.BlockSpec((T * Cf, 1), lambda b, j: (0, 0)),
            pl.BlockSpec((Cf, 1), lambda b, j: (0, 0)),
            pl.BlockSpec((1, C * T, hw_tile), lambda b, j: (b, 0, j)),
        ],
        out_specs=pl.BlockSpec((1, Cf, 1), lambda b, j: (b, 0, 0)),
        scratch_shapes=(
            [] if n_hw == 1
            else [pltpu.VMEM((Cf, hw_tile), jnp.float32)]),
        compiler_params=pltpu.CompilerParams(
            dimension_semantics=("parallel", "arbitrary"),
            vmem_limit_bytes=_VMEM_LIMIT,
        ),
    )(wcat, nb_big, b_bb, x3)
    return out[:, :, 0]


def _head(ctx, roi2, pb8, mask_col, wr, wg8, wc, b1, wcls, bcls, nbox):
    M, Dr = roi2.shape
    B, Cf = ctx.shape
    Hh = wr.shape[1]
    ncls_pad = wcls.shape[1]
    n_split = 2
    half = M // n_split
    b_half = B // n_split

    def body(ctx_ref, roi_ref, pb_ref, m_ref, wr_ref, wg_ref, wc_ref,
             b1_ref, wcls_ref, bcls_ref, out_ref):
        i = pl.program_id(0)
        ctxh = jnp.dot(ctx_ref[...], wc_ref[...],
                       preferred_element_type=jnp.float32) + b1_ref[...]
        row_b = jax.lax.broadcasted_iota(jnp.int32, (half, B), 0) // nbox
        col_b = jax.lax.broadcasted_iota(jnp.int32, (half, B), 1)
        sel = (row_b + i * b_half == col_b).astype(jnp.float32)
        h = jnp.dot(roi_ref[...], wr_ref[...],
                    preferred_element_type=jnp.float32)
        h = h + jnp.dot(pb_ref[...], wg_ref[...],
                        preferred_element_type=jnp.float32)
        h = h + jnp.dot(sel, ctxh, preferred_element_type=jnp.float32)
        h = jnp.maximum(h, 0.0)
        logits = jnp.dot(h, wcls_ref[...],
                         preferred_element_type=jnp.float32) + bcls_ref[...]
        out_ref[...] = logits * m_ref[...]

    return pl.pallas_call(
        body,
        out_shape=jax.ShapeDtypeStruct((M, ncls_pad), jnp.float32),
        grid=(n_split,),
        in_specs=[
            pl.BlockSpec((B, Cf), lambda i: (0, 0)),
            pl.BlockSpec((half, Dr), lambda i: (i, 0)),
            pl.BlockSpec((half, 8), lambda i: (i, 0)),
            pl.BlockSpec((half, 1), lambda i: (i, 0)),
            pl.BlockSpec((Dr, Hh), lambda i: (0, 0)),
            pl.BlockSpec((8, Hh), lambda i: (0, 0)),
            pl.BlockSpec((Cf, Hh), lambda i: (0, 0)),
            pl.BlockSpec((1, Hh), lambda i: (0, 0)),
            pl.BlockSpec((Hh, ncls_pad), lambda i: (0, 0)),
            pl.BlockSpec((1, ncls_pad), lambda i: (0, 0)),
        ],
        out_specs=pl.BlockSpec((half, ncls_pad), lambda i: (i, 0)),
        compiler_params=pltpu.CompilerParams(
            dimension_semantics=("parallel",),
            vmem_limit_bytes=_VMEM_LIMIT,
        ),
    )(ctx, roi2, pb8, mask_col, wr, wg8, wc, b1, wcls, bcls)


def kernel(clips, pred_boxes, roi_features, pred_scores, pred_boxes_mask,
           w_bb, b_bb, wc, wr, wg, b1, wcls, bcls):
    B, nbox, dr = roi_features.shape
    ncls_pad = wcls.shape[1]
    M = B * nbox

    ctx = _backbone_ctx(clips, w_bb, b_bb)

    gmap = jnp.array(
        [[1, 0, 0, 0, 0, -1],
         [0, 1, 0, 0, -1, 0],
         [0, 0, 1, 0, 0, 1],
         [0, 0, 0, 1, 1, 0]], dtype=wg.dtype)
    wg_eff = gmap @ wg
    wg8 = jnp.concatenate(
        [wg_eff, jnp.zeros((4, wg.shape[1]), wg.dtype)], axis=0)
    pb8 = jnp.concatenate(
        [pred_boxes.reshape(M, 4),
         jnp.zeros((M, 4), pred_boxes.dtype)], axis=1)
    roi2 = roi_features.reshape(M, dr)
    mask_col = pred_boxes_mask.astype(jnp.float32).reshape(M, 1)

    logits = _head(ctx, roi2, pb8, mask_col, wr, wg8, wc, b1, wcls, bcls,
                   nbox)
    pred_logits = logits.reshape(B, nbox, ncls_pad)[:, :, :_NUM_CLASSES]

    outputs = {
        "pred_logits": pred_logits,
        "pred_scores": pred_scores,
        "pred_boxes_mask": pred_boxes_mask,
        "pred_boxes": pred_boxes,
    }
    return {"outputs": outputs}
